# baseline (speedup 1.0000x reference)
import functools

import jax
import jax.numpy as jnp
from jax import lax
from jax.experimental import pallas as pl
from jax.experimental.pallas import tpu as pltpu
from jax.experimental.pallas import tpu_sc as plsc

N = 10000
E = 320000
U = 32
NC = 2
NS = 16
NW = NC * NS
EW = E // NW
C = 400
NCH = EW // C
NP = 10240
RPS = NP // NS
E4 = E // 4
BM = 8000
GE = E4 // BM
EPS = 1e-5

_f32 = jnp.float32


def _sc_layer_body(w0, x2t, x3t, x4t, src, dst, zt,
                   aggp, ew,
                   src_v0, src_v1, dst_v0, dst_v1,
                   w0_v0, w0_v1, x2r0, x2r1, x3r0, x3r1, x4r0, x4r1,
                   agg_s, sem_idx, semg0, semg1):
    src_v = (src_v0, src_v1)
    dst_v = (dst_v0, dst_v1)
    w0_v = (w0_v0, w0_v1)
    x2r = (x2r0, x2r1)
    x3r = (x3r0, x3r1)
    x4r = (x4r0, x4r1)
    semg = (semg0, semg1)

    c = lax.axis_index("c")
    s = lax.axis_index("s")
    wid = c * NS + s
    pltpu.sync_copy(zt.at[pl.ds(s * RPS, RPS)], agg_s.at[pl.ds(s * RPS, RPS)])
    plsc.subcore_barrier()

    def _issue(p, nb):
        base = pl.multiple_of(wid * EW + p * C, 8)
        ci1 = pltpu.async_copy(src.at[pl.ds(base, C)], src_v[nb], sem_idx)
        ci2 = pltpu.async_copy(dst.at[pl.ds(base, C)], dst_v[nb], sem_idx)
        ci1.wait()
        ci2.wait()
        pltpu.async_copy(w0.at[pl.ds(base, C)], w0_v[nb], semg[nb])
        pltpu.async_copy(x2t.at[dst_v[nb]], x2r[nb], semg[nb])
        pltpu.async_copy(x3t.at[src_v[nb]], x3r[nb], semg[nb])
        pltpu.async_copy(x4t.at[dst_v[nb]], x4r[nb], semg[nb])

    def _iter(p, b, issue_next):
        if issue_next:
            _issue(p + 1, 1 - b)
        pltpu.make_async_copy(w0.at[pl.ds(0, C)], w0_v[b], semg[b]).wait()
        pltpu.make_async_copy(w0.at[pl.ds(0, C)], x2r[b], semg[b]).wait()
        pltpu.make_async_copy(w0.at[pl.ds(0, C)], x3r[b], semg[b]).wait()
        pltpu.make_async_copy(w0.at[pl.ds(0, C)], x4r[b], semg[b]).wait()

        def row(r2, rc):
            for u in range(8):
                r = r2 * 8 + u
                for h in (0, 16):
                    wv = w0_v[b][r, pl.ds(h, 16)]
                    sig = 1.0 / (1.0 + jnp.exp(-wv))
                    x2r[b][r, pl.ds(h, 16)] = sig * x2r[b][r, pl.ds(h, 16)]
                    x3r[b][r, pl.ds(h, 16)] = (x3r[b][r, pl.ds(h, 16)]
                                               + x4r[b][r, pl.ds(h, 16)])
            return rc

        lax.fori_loop(0, C // 8, row, 0)
        base = pl.multiple_of(wid * EW + p * C, 8)
        pltpu.sync_copy(x2r[b], agg_s.at[src_v[b]], add=True)
        pltpu.sync_copy(x3r[b], ew.at[pl.ds(base, C)])

    _issue(0, 0)
    _iter(0, 0, True)

    def pair(q, carry):
        _iter(1 + 2 * q, 1, True)
        _iter(2 + 2 * q, 0, True)
        return carry

    lax.fori_loop(0, (NCH - 3) // 2, pair, 0)
    _iter(NCH - 2, 1, True)
    _iter(NCH - 1, 0, False)
    plsc.subcore_barrier()
    pltpu.sync_copy(agg_s.at[pl.ds(s * RPS, RPS)],
                    aggp.at[c, pl.ds(s * RPS, RPS)])


_sc_layer = functools.partial(
    pl.kernel,
    mesh=plsc.VectorSubcoreMesh(core_axis_name="c", subcore_axis_name="s"),
    compiler_params=pltpu.CompilerParams(use_tc_tiling_on_sc=False),
    out_type=[jax.ShapeDtypeStruct((NC, NP, U), _f32),
              jax.ShapeDtypeStruct((E, U), _f32)],
    scratch_types=(
        [pltpu.VMEM((C,), jnp.int32)] * 4
        + [pltpu.VMEM((C, U), _f32)] * 8
        + [pltpu.VMEM_SHARED((NP, U), _f32)]
        + [pltpu.SemaphoreType.DMA] * 3
    ),
)(_sc_layer_body)


def _sc_count_body(src, zt, on1, cntp, src_v, ones_v, cnt_s, sem):
    c = lax.axis_index("c")
    s = lax.axis_index("s")
    wid = c * NS + s
    pltpu.sync_copy(zt.at[pl.ds(s * RPS, RPS)], cnt_s.at[pl.ds(s * RPS, RPS)])
    pltpu.sync_copy(on1, ones_v)
    plsc.subcore_barrier()

    def chunk(k, carry):
        base = pl.multiple_of(wid * EW + k * C, 8)
        pltpu.sync_copy(src.at[pl.ds(base, C)], src_v)
        pltpu.sync_copy(ones_v, cnt_s.at[src_v], add=True)
        return carry

    lax.fori_loop(0, NCH, chunk, 0)
    plsc.subcore_barrier()
    pltpu.sync_copy(cnt_s.at[pl.ds(s * RPS, RPS)],
                    cntp.at[c, pl.ds(s * RPS, RPS)])


_sc_count = functools.partial(
    pl.kernel,
    mesh=plsc.VectorSubcoreMesh(core_axis_name="c", subcore_axis_name="s"),
    compiler_params=pltpu.CompilerParams(use_tc_tiling_on_sc=False),
    out_type=[jax.ShapeDtypeStruct((NC, NP, U), _f32)],
    scratch_types=[
        pltpu.VMEM((C,), jnp.int32),
        pltpu.VMEM((C, U), _f32),
        pltpu.VMEM_SHARED((NP, U), _f32),
        pltpu.SemaphoreType.DMA,
    ],
)(_sc_count_body)


def _silu(t):
    return t * (1.0 / (1.0 + jnp.exp(-t)))


def _init_x_body(x_ref, w_ref, b_ref, o_ref):
    t = x_ref[...] * w_ref[...] + b_ref[...]
    o_ref[...] = _silu(t)


def _tc_init_x(x, Wv0, bv0):
    return pl.pallas_call(
        _init_x_body,
        out_shape=jax.ShapeDtypeStruct((N, U), _f32),
    )(x, Wv0, bv0.reshape(1, U))


def _init_w_body(ea_ref, w_ref, b_ref, o_ref):
    t = jnp.dot(ea_ref[...], w_ref[...], preferred_element_type=_f32)
    o_ref[...] = _silu(t + b_ref[...])


def _tc_init_w(ea4, We0bd, be0t):
    return pl.pallas_call(
        _init_w_body,
        grid=(GE,),
        in_specs=[pl.BlockSpec((BM, 12), lambda i: (i, 0)),
                  pl.BlockSpec((12, 128), lambda i: (0, 0)),
                  pl.BlockSpec((1, 128), lambda i: (0, 0))],
        out_specs=pl.BlockSpec((BM, 128), lambda i: (i, 0)),
        out_shape=jax.ShapeDtypeStruct((E4, 128), _f32),
    )(ea4, We0bd, be0t)


def _node_proj_body(x_ref, w1, b1, w2, b2, w3, b3, w4, b4, o1, o2, o3, o4):
    xv = x_ref[...]
    o1[...] = jnp.dot(xv, w1[...], preferred_element_type=_f32) + b1[...]
    o2[...] = jnp.dot(xv, w2[...], preferred_element_type=_f32) + b2[...]
    o3[...] = jnp.dot(xv, w3[...], preferred_element_type=_f32) + b3[...]
    o4[...] = jnp.dot(xv, w4[...], preferred_element_type=_f32) + b4[...]


def _tc_node_proj(x, w1, b1, w2, b2, w3, b3, w4, b4):
    sh = jax.ShapeDtypeStruct((N, U), _f32)
    return pl.pallas_call(
        _node_proj_body,
        out_shape=[sh, sh, sh, sh],
    )(x, w1, b1.reshape(1, U), w2, b2.reshape(1, U),
      w3, b3.reshape(1, U), w4, b4.reshape(1, U))


def _node_upd_body(x_ref, x1_ref, aggp_ref, cntp_ref, g_ref, b_ref, o_ref):
    cnt = jnp.maximum(cntp_ref[0, :N, :1] + cntp_ref[1, :N, :1], 1.0)
    agg = (aggp_ref[0, :N, :] + aggp_ref[1, :N, :]) / cnt
    t = x1_ref[...] + agg
    m = jnp.mean(t, axis=0, keepdims=True)
    v = jnp.mean((t - m) * (t - m), axis=0, keepdims=True)
    tn = (t - m) / jnp.sqrt(v + EPS) * g_ref[...] + b_ref[...]
    o_ref[...] = x_ref[...] + _silu(tn)


def _tc_node_update(x, x1, aggp, cntp, g, b):
    return pl.pallas_call(
        _node_upd_body,
        out_shape=jax.ShapeDtypeStruct((N, U), _f32),
    )(x, x1, aggp, cntp, g.reshape(1, U), b.reshape(1, U))


def _invcnt_body(cntp_ref, o_ref):
    cnt = jnp.maximum(cntp_ref[0, :N, :] + cntp_ref[1, :N, :], 1.0)
    o_ref[...] = 1.0 / cnt


def _tc_invcnt(cntp):
    return pl.pallas_call(
        _invcnt_body,
        out_shape=jax.ShapeDtypeStruct((N, U), _f32),
    )(cntp)


def _node_step_body(x_ref, x1_ref, aggp_ref, invc_ref, g_ref, b_ref,
                    w1, b1, w2, b2, w3, b3, w4, b4,
                    ox, o1, o2, o3, o4):
    agg = (aggp_ref[0, :N, :] + aggp_ref[1, :N, :]) * invc_ref[...]
    t = x1_ref[...] + agg
    m = jnp.mean(t, axis=0, keepdims=True)
    v = jnp.mean((t - m) * (t - m), axis=0, keepdims=True)
    tn = (t - m) / jnp.sqrt(v + EPS) * g_ref[...] + b_ref[...]
    xn = x_ref[...] + _silu(tn)
    ox[...] = xn
    o1[...] = jnp.dot(xn, w1[...], preferred_element_type=_f32) + b1[...]
    o2[...] = jnp.dot(xn, w2[...], preferred_element_type=_f32) + b2[...]
    o3[...] = jnp.dot(xn, w3[...], preferred_element_type=_f32) + b3[...]
    o4[...] = jnp.dot(xn, w4[...], preferred_element_type=_f32) + b4[...]


def _tc_node_step(x, x1, aggp, invc, g, b, w1, b1, w2, b2, w3, b3, w4, b4):
    sh = jax.ShapeDtypeStruct((N, U), _f32)
    return pl.pallas_call(
        _node_step_body,
        out_shape=[sh, sh, sh, sh, sh],
    )(x, x1, aggp, invc, g.reshape(1, U), b.reshape(1, U),
      w1, b1.reshape(1, U), w2, b2.reshape(1, U),
      w3, b3.reshape(1, U), w4, b4.reshape(1, U))


def _edge_tail_body(w_ref, t_ref, mom_ref, fold_ref, g_ref, b_ref,
                    wp0, bp0r, wp1, bp1r, wp2, bp2r, o_ref):
    s = jnp.dot(mom_ref[...], fold_ref[...], preferred_element_type=_f32)
    m = s[0:1, :] / E
    v = s[1:2, :] / E - m * m
    a = g_ref[...] / jnp.sqrt(v + EPS)
    cte = b_ref[...] - m * a
    ft = jnp.transpose(fold_ref[...], (1, 0))
    a128 = jnp.dot(a, ft, preferred_element_type=_f32)
    c128 = jnp.dot(cte, ft, preferred_element_type=_f32)
    wn = w_ref[...] + _silu(t_ref[...] * a128 + c128)
    h = _silu(jnp.dot(wn, wp0[...], preferred_element_type=_f32) + bp0r[...])
    h = _silu(jnp.dot(h, wp1[...], preferred_element_type=_f32) + bp1r[...])
    t = jnp.dot(h, wp2[...], preferred_element_type=_f32) + bp2r[...]
    o_ref[...] = 1.0 / (1.0 + jnp.exp(-t))


def _tc_edge_tail(w4, t4, mom, fold, g, b, Wp0bd, bp0t, Wp1bd, bp1t, Wp2bd, bp2t):
    return pl.pallas_call(
        _edge_tail_body,
        grid=(GE,),
        in_specs=[pl.BlockSpec((BM, 128), lambda i: (i, 0)),
                  pl.BlockSpec((BM, 128), lambda i: (i, 0)),
                  pl.BlockSpec((2, 128), lambda i: (0, 0)),
                  pl.BlockSpec((128, U), lambda i: (0, 0)),
                  pl.BlockSpec((1, U), lambda i: (0, 0)),
                  pl.BlockSpec((1, U), lambda i: (0, 0)),
                  pl.BlockSpec((128, 128), lambda i: (0, 0)),
                  pl.BlockSpec((1, 128), lambda i: (0, 0)),
                  pl.BlockSpec((128, 128), lambda i: (0, 0)),
                  pl.BlockSpec((1, 128), lambda i: (0, 0)),
                  pl.BlockSpec((128, 4), lambda i: (0, 0)),
                  pl.BlockSpec((1, 4), lambda i: (0, 0))],
        out_specs=pl.BlockSpec((BM, 4), lambda i: (i, 0)),
        out_shape=jax.ShapeDtypeStruct((E4, 4), _f32),
    )(w4, t4, mom, fold, g.reshape(1, U), b.reshape(1, U),
      Wp0bd, bp0t, Wp1bd, bp1t, Wp2bd, bp2t)


def _edge_mm_body(w_ref, ew_ref, wbd_ref, b_ref, t_ref, mom_ref):
    i = pl.program_id(0)
    t = (jnp.dot(w_ref[...], wbd_ref[...], preferred_element_type=_f32)
         + b_ref[...] + ew_ref[...])
    t_ref[...] = t
    s1 = jnp.sum(t, axis=0, keepdims=True)
    s2 = jnp.sum(t * t, axis=0, keepdims=True)
    blk = jnp.concatenate([s1, s2], axis=0)

    @pl.when(i == 0)
    def _():
        mom_ref[...] = blk

    @pl.when(i != 0)
    def _():
        mom_ref[...] = mom_ref[...] + blk


def _tc_edge_mm(w4, ew4, Wbd, b128):
    return pl.pallas_call(
        _edge_mm_body,
        grid=(GE,),
        in_specs=[pl.BlockSpec((BM, 128), lambda i: (i, 0)),
                  pl.BlockSpec((BM, 128), lambda i: (i, 0)),
                  pl.BlockSpec((128, 128), lambda i: (0, 0)),
                  pl.BlockSpec((1, 128), lambda i: (0, 0))],
        out_specs=[pl.BlockSpec((BM, 128), lambda i: (i, 0)),
                   pl.BlockSpec((2, 128), lambda i: (0, 0))],
        out_shape=[jax.ShapeDtypeStruct((E4, 128), _f32),
                   jax.ShapeDtypeStruct((2, 128), _f32)],
    )(w4, ew4, Wbd, b128)


def _edge_upd_body(w_ref, t_ref, mom_ref, fold_ref, g_ref, b_ref, o_ref):
    s = jnp.dot(mom_ref[...], fold_ref[...], preferred_element_type=_f32)
    m = s[0:1, :] / E
    v = s[1:2, :] / E - m * m
    a = g_ref[...] / jnp.sqrt(v + EPS)
    cte = b_ref[...] - m * a
    ft = jnp.transpose(fold_ref[...], (1, 0))
    a128 = jnp.dot(a, ft, preferred_element_type=_f32)
    c128 = jnp.dot(cte, ft, preferred_element_type=_f32)
    tn = t_ref[...] * a128 + c128
    o_ref[...] = w_ref[...] + _silu(tn)


def _tc_edge_update(w4, t4, mom, fold, g, b):
    return pl.pallas_call(
        _edge_upd_body,
        grid=(GE,),
        in_specs=[pl.BlockSpec((BM, 128), lambda i: (i, 0)),
                  pl.BlockSpec((BM, 128), lambda i: (i, 0)),
                  pl.BlockSpec((2, 128), lambda i: (0, 0)),
                  pl.BlockSpec((128, U), lambda i: (0, 0)),
                  pl.BlockSpec((1, U), lambda i: (0, 0)),
                  pl.BlockSpec((1, U), lambda i: (0, 0))],
        out_specs=pl.BlockSpec((BM, 128), lambda i: (i, 0)),
        out_shape=jax.ShapeDtypeStruct((E4, 128), _f32),
    )(w4, t4, mom, fold, g.reshape(1, U), b.reshape(1, U))


def _head_body(w_ref, w0_ref, b0_ref, w1_ref, b1_ref, w2_ref, b2_ref, o_ref):
    h = _silu(jnp.dot(w_ref[...], w0_ref[...], preferred_element_type=_f32)
              + b0_ref[...])
    h = _silu(jnp.dot(h, w1_ref[...], preferred_element_type=_f32)
              + b1_ref[...])
    t = jnp.dot(h, w2_ref[...], preferred_element_type=_f32) + b2_ref[...]
    o_ref[...] = 1.0 / (1.0 + jnp.exp(-t))


def _tc_head(w4, Wp0bd, bp0t, Wp1bd, bp1t, Wp2bd, bp2t):
    return pl.pallas_call(
        _head_body,
        grid=(GE,),
        in_specs=[pl.BlockSpec((BM, 128), lambda i: (i, 0)),
                  pl.BlockSpec((128, 128), lambda i: (0, 0)),
                  pl.BlockSpec((1, 128), lambda i: (0, 0)),
                  pl.BlockSpec((128, 128), lambda i: (0, 0)),
                  pl.BlockSpec((1, 128), lambda i: (0, 0)),
                  pl.BlockSpec((128, 4), lambda i: (0, 0)),
                  pl.BlockSpec((1, 4), lambda i: (0, 0))],
        out_specs=pl.BlockSpec((BM, 4), lambda i: (i, 0)),
        out_shape=jax.ShapeDtypeStruct((E4, 4), _f32),
    )(w4, Wp0bd, bp0t, Wp1bd, bp1t, Wp2bd, bp2t)


def _bd4(w):
    return jnp.kron(jnp.eye(4, dtype=_f32), w)


def kernel(x, edge_index, edge_attr, Wv0, bv0, We0, be0, Wv1, bv1, Wv2, bv2,
           Wv3, bv3, Wv4, bv4, We1, be1, vbn_g, vbn_b, ebn_g, ebn_b,
           Wp0, bp0, Wp1, bp1, Wp2, bp2):
    src = edge_index[0]
    dst = edge_index[1]
    zt = jnp.zeros((NP, U), dtype=_f32)
    on1 = jnp.ones((C, U), dtype=_f32)
    fold = jnp.tile(jnp.eye(U, dtype=_f32), (4, 1))

    x = _tc_init_x(x, Wv0, bv0)
    ea4 = edge_attr.reshape(E4, 12)
    w4 = _tc_init_w(ea4, _bd4(We0), jnp.tile(be0, 4).reshape(1, 128))
    (cntp,) = _sc_count(src, zt, on1)
    invc = _tc_invcnt(cntp)

    We1bd = jax.vmap(_bd4)(We1)
    be1t = jnp.tile(be1, (1, 4)).reshape(-1, 1, 128)

    D = Wv1.shape[0]
    x1, x2, x3, x4 = _tc_node_proj(x, Wv1[0], bv1[0], Wv2[0], bv2[0],
                                   Wv3[0], bv3[0], Wv4[0], bv4[0])
    for i in range(D):
        aggp, ewg = _sc_layer(w4.reshape(E, U), x2, x3, x4, src, dst, zt)
        if i + 1 < D:
            x, x1, x2, x3, x4 = _tc_node_step(
                x, x1, aggp, invc, vbn_g[i], vbn_b[i],
                Wv1[i + 1], bv1[i + 1], Wv2[i + 1], bv2[i + 1],
                Wv3[i + 1], bv3[i + 1], Wv4[i + 1], bv4[i + 1])
        t4, mom = _tc_edge_mm(w4, ewg.reshape(E4, 128), We1bd[i], be1t[i])
        if i + 1 < D:
            w4 = _tc_edge_update(w4, t4, mom, fold, ebn_g[i], ebn_b[i])
    heu4 = _tc_edge_tail(w4, t4, mom, fold, ebn_g[D - 1], ebn_b[D - 1],
                         _bd4(Wp0), jnp.tile(bp0, 4).reshape(1, 128),
                         _bd4(Wp1), jnp.tile(bp1, 4).reshape(1, 128),
                         _bd4(Wp2), jnp.tile(bp2, 4).reshape(1, 4))
    return heu4.reshape(E)


# speedup vs baseline: 1.0519x; 1.0519x over previous
import functools

import jax
import jax.numpy as jnp
from jax import lax
from jax.experimental import pallas as pl
from jax.experimental.pallas import tpu as pltpu
from jax.experimental.pallas import tpu_sc as plsc

N = 10000
E = 320000
U = 32
NC = 2
NS = 16
NW = NC * NS
EW = E // NW
C = 400
NCH = EW // C
NP = 10240
RPS = NP // NS
E4 = E // 4
BM = 8000
GE = E4 // BM
EPS = 1e-5

_f32 = jnp.float32


def _sc_layer_body(w0, x2t, x3t, x4t, src, dst, zt,
                   aggp, ew,
                   src_v0, src_v1, dst_v0, dst_v1,
                   w0_v0, w0_v1, x2r0, x2r1, x3r0, x3r1, x4r0, x4r1,
                   agg_s, sem_idx, semg0, semg1):
    src_v = (src_v0, src_v1)
    dst_v = (dst_v0, dst_v1)
    w0_v = (w0_v0, w0_v1)
    x2r = (x2r0, x2r1)
    x3r = (x3r0, x3r1)
    x4r = (x4r0, x4r1)
    semg = (semg0, semg1)

    c = lax.axis_index("c")
    s = lax.axis_index("s")
    wid = c * NS + s
    pltpu.sync_copy(zt.at[pl.ds(s * RPS, RPS)], agg_s.at[pl.ds(s * RPS, RPS)])
    plsc.subcore_barrier()

    def _issue(p, nb):
        base = pl.multiple_of(wid * EW + p * C, 8)
        ci1 = pltpu.async_copy(src.at[pl.ds(base, C)], src_v[nb], sem_idx)
        ci2 = pltpu.async_copy(dst.at[pl.ds(base, C)], dst_v[nb], sem_idx)
        ci1.wait()
        ci2.wait()
        pltpu.async_copy(w0.at[pl.ds(base, C)], w0_v[nb], semg[nb])
        pltpu.async_copy(x2t.at[dst_v[nb]], x2r[nb], semg[nb])
        pltpu.async_copy(x3t.at[src_v[nb]], x3r[nb], semg[nb])
        pltpu.async_copy(x4t.at[dst_v[nb]], x4r[nb], semg[nb])

    def _iter(p, b, issue_next):
        if issue_next:
            _issue(p + 1, 1 - b)
        pltpu.make_async_copy(w0.at[pl.ds(0, C)], w0_v[b], semg[b]).wait()
        pltpu.make_async_copy(w0.at[pl.ds(0, C)], x2r[b], semg[b]).wait()
        pltpu.make_async_copy(w0.at[pl.ds(0, C)], x3r[b], semg[b]).wait()
        pltpu.make_async_copy(w0.at[pl.ds(0, C)], x4r[b], semg[b]).wait()

        def row(r2, rc):
            for u in range(4):
                r = r2 * 4 + u
                for h in (0, 16):
                    wv = w0_v[b][r, pl.ds(h, 16)]
                    sig = 1.0 / (1.0 + jnp.exp(-wv))
                    x2r[b][r, pl.ds(h, 16)] = sig * x2r[b][r, pl.ds(h, 16)]
                    x3r[b][r, pl.ds(h, 16)] = (x3r[b][r, pl.ds(h, 16)]
                                               + x4r[b][r, pl.ds(h, 16)])
            return rc

        lax.fori_loop(0, C // 4, row, 0)
        base = pl.multiple_of(wid * EW + p * C, 8)
        pltpu.sync_copy(x2r[b], agg_s.at[src_v[b]], add=True)
        pltpu.sync_copy(x3r[b], ew.at[pl.ds(base, C)])

    _issue(0, 0)
    _iter(0, 0, True)

    def pair(q, carry):
        _iter(1 + 2 * q, 1, True)
        _iter(2 + 2 * q, 0, True)
        return carry

    lax.fori_loop(0, (NCH - 3) // 2, pair, 0)
    _iter(NCH - 2, 1, True)
    _iter(NCH - 1, 0, False)
    plsc.subcore_barrier()
    pltpu.sync_copy(agg_s.at[pl.ds(s * RPS, RPS)],
                    aggp.at[c, pl.ds(s * RPS, RPS)])


_sc_layer = functools.partial(
    pl.kernel,
    mesh=plsc.VectorSubcoreMesh(core_axis_name="c", subcore_axis_name="s"),
    compiler_params=pltpu.CompilerParams(use_tc_tiling_on_sc=False),
    out_type=[jax.ShapeDtypeStruct((NC, NP, U), _f32),
              jax.ShapeDtypeStruct((E, U), _f32)],
    scratch_types=(
        [pltpu.VMEM((C,), jnp.int32)] * 4
        + [pltpu.VMEM((C, U), _f32)] * 8
        + [pltpu.VMEM_SHARED((NP, U), _f32)]
        + [pltpu.SemaphoreType.DMA] * 3
    ),
)(_sc_layer_body)


def _sc_count_body(src, zt, on1, cntp, src_v, ones_v, cnt_s, sem):
    c = lax.axis_index("c")
    s = lax.axis_index("s")
    wid = c * NS + s
    pltpu.sync_copy(zt.at[pl.ds(s * RPS, RPS)], cnt_s.at[pl.ds(s * RPS, RPS)])
    pltpu.sync_copy(on1, ones_v)
    plsc.subcore_barrier()

    def chunk(k, carry):
        base = pl.multiple_of(wid * EW + k * C, 8)
        pltpu.sync_copy(src.at[pl.ds(base, C)], src_v)
        pltpu.sync_copy(ones_v, cnt_s.at[src_v], add=True)
        return carry

    lax.fori_loop(0, NCH, chunk, 0)
    plsc.subcore_barrier()
    pltpu.sync_copy(cnt_s.at[pl.ds(s * RPS, RPS)],
                    cntp.at[c, pl.ds(s * RPS, RPS)])


_sc_count = functools.partial(
    pl.kernel,
    mesh=plsc.VectorSubcoreMesh(core_axis_name="c", subcore_axis_name="s"),
    compiler_params=pltpu.CompilerParams(use_tc_tiling_on_sc=False),
    out_type=[jax.ShapeDtypeStruct((NC, NP, U), _f32)],
    scratch_types=[
        pltpu.VMEM((C,), jnp.int32),
        pltpu.VMEM((C, U), _f32),
        pltpu.VMEM_SHARED((NP, U), _f32),
        pltpu.SemaphoreType.DMA,
    ],
)(_sc_count_body)


def _silu(t):
    return t * (1.0 / (1.0 + jnp.exp(-t)))


def _init_x_body(x_ref, w_ref, b_ref, o_ref):
    t = x_ref[...] * w_ref[...] + b_ref[...]
    o_ref[...] = _silu(t)


def _tc_init_x(x, Wv0, bv0):
    return pl.pallas_call(
        _init_x_body,
        out_shape=jax.ShapeDtypeStruct((N, U), _f32),
    )(x, Wv0, bv0.reshape(1, U))


def _init_w_body(ea_ref, w_ref, b_ref, o_ref):
    t = jnp.dot(ea_ref[...], w_ref[...], preferred_element_type=_f32)
    o_ref[...] = _silu(t + b_ref[...])


def _tc_init_w(ea4, We0bd, be0t):
    return pl.pallas_call(
        _init_w_body,
        grid=(GE,),
        in_specs=[pl.BlockSpec((BM, 12), lambda i: (i, 0)),
                  pl.BlockSpec((12, 128), lambda i: (0, 0)),
                  pl.BlockSpec((1, 128), lambda i: (0, 0))],
        out_specs=pl.BlockSpec((BM, 128), lambda i: (i, 0)),
        out_shape=jax.ShapeDtypeStruct((E4, 128), _f32),
    )(ea4, We0bd, be0t)


def _node_proj_body(x_ref, w1, b1, w2, b2, w3, b3, w4, b4, o1, o2, o3, o4):
    xv = x_ref[...]
    o1[...] = jnp.dot(xv, w1[...], preferred_element_type=_f32) + b1[...]
    o2[...] = jnp.dot(xv, w2[...], preferred_element_type=_f32) + b2[...]
    o3[...] = jnp.dot(xv, w3[...], preferred_element_type=_f32) + b3[...]
    o4[...] = jnp.dot(xv, w4[...], preferred_element_type=_f32) + b4[...]


def _tc_node_proj(x, w1, b1, w2, b2, w3, b3, w4, b4):
    sh = jax.ShapeDtypeStruct((N, U), _f32)
    return pl.pallas_call(
        _node_proj_body,
        out_shape=[sh, sh, sh, sh],
    )(x, w1, b1.reshape(1, U), w2, b2.reshape(1, U),
      w3, b3.reshape(1, U), w4, b4.reshape(1, U))


def _node_upd_body(x_ref, x1_ref, aggp_ref, cntp_ref, g_ref, b_ref, o_ref):
    cnt = jnp.maximum(cntp_ref[0, :N, :1] + cntp_ref[1, :N, :1], 1.0)
    agg = (aggp_ref[0, :N, :] + aggp_ref[1, :N, :]) / cnt
    t = x1_ref[...] + agg
    m = jnp.mean(t, axis=0, keepdims=True)
    v = jnp.mean((t - m) * (t - m), axis=0, keepdims=True)
    tn = (t - m) / jnp.sqrt(v + EPS) * g_ref[...] + b_ref[...]
    o_ref[...] = x_ref[...] + _silu(tn)


def _tc_node_update(x, x1, aggp, cntp, g, b):
    return pl.pallas_call(
        _node_upd_body,
        out_shape=jax.ShapeDtypeStruct((N, U), _f32),
    )(x, x1, aggp, cntp, g.reshape(1, U), b.reshape(1, U))


def _invcnt_body(cntp_ref, o_ref):
    cnt = jnp.maximum(cntp_ref[0, :N, :] + cntp_ref[1, :N, :], 1.0)
    o_ref[...] = 1.0 / cnt


def _tc_invcnt(cntp):
    return pl.pallas_call(
        _invcnt_body,
        out_shape=jax.ShapeDtypeStruct((N, U), _f32),
    )(cntp)


def _node_step_body(x_ref, x1_ref, aggp_ref, invc_ref, g_ref, b_ref,
                    w1, b1, w2, b2, w3, b3, w4, b4,
                    ox, o1, o2, o3, o4):
    agg = (aggp_ref[0, :N, :] + aggp_ref[1, :N, :]) * invc_ref[...]
    t = x1_ref[...] + agg
    m = jnp.mean(t, axis=0, keepdims=True)
    v = jnp.mean((t - m) * (t - m), axis=0, keepdims=True)
    tn = (t - m) / jnp.sqrt(v + EPS) * g_ref[...] + b_ref[...]
    xn = x_ref[...] + _silu(tn)
    ox[...] = xn
    o1[...] = jnp.dot(xn, w1[...], preferred_element_type=_f32) + b1[...]
    o2[...] = jnp.dot(xn, w2[...], preferred_element_type=_f32) + b2[...]
    o3[...] = jnp.dot(xn, w3[...], preferred_element_type=_f32) + b3[...]
    o4[...] = jnp.dot(xn, w4[...], preferred_element_type=_f32) + b4[...]


def _tc_node_step(x, x1, aggp, invc, g, b, w1, b1, w2, b2, w3, b3, w4, b4):
    sh = jax.ShapeDtypeStruct((N, U), _f32)
    return pl.pallas_call(
        _node_step_body,
        out_shape=[sh, sh, sh, sh, sh],
    )(x, x1, aggp, invc, g.reshape(1, U), b.reshape(1, U),
      w1, b1.reshape(1, U), w2, b2.reshape(1, U),
      w3, b3.reshape(1, U), w4, b4.reshape(1, U))


def _edge_tail_body(w_ref, t_ref, mom_ref, fold_ref, g_ref, b_ref,
                    wp0, bp0r, wp1, bp1r, wp2, bp2r, o_ref):
    s = jnp.dot(mom_ref[...], fold_ref[...], preferred_element_type=_f32)
    m = s[0:1, :] / E
    v = s[1:2, :] / E - m * m
    a = g_ref[...] / jnp.sqrt(v + EPS)
    cte = b_ref[...] - m * a
    ft = jnp.transpose(fold_ref[...], (1, 0))
    a128 = jnp.dot(a, ft, preferred_element_type=_f32)
    c128 = jnp.dot(cte, ft, preferred_element_type=_f32)
    wn = w_ref[...] + _silu(t_ref[...].astype(_f32) * a128 + c128)
    h = _silu(jnp.dot(wn, wp0[...], preferred_element_type=_f32) + bp0r[...])
    h = _silu(jnp.dot(h, wp1[...], preferred_element_type=_f32) + bp1r[...])
    t = jnp.dot(h, wp2[...], preferred_element_type=_f32) + bp2r[...]
    o_ref[...] = 1.0 / (1.0 + jnp.exp(-t))


def _tc_edge_tail(w4, t4, mom, fold, g, b, Wp0bd, bp0t, Wp1bd, bp1t, Wp2bd, bp2t):
    return pl.pallas_call(
        _edge_tail_body,
        grid=(GE,),
        in_specs=[pl.BlockSpec((BM, 128), lambda i: (i, 0)),
                  pl.BlockSpec((BM, 128), lambda i: (i, 0)),
                  pl.BlockSpec((2, 128), lambda i: (0, 0)),
                  pl.BlockSpec((128, U), lambda i: (0, 0)),
                  pl.BlockSpec((1, U), lambda i: (0, 0)),
                  pl.BlockSpec((1, U), lambda i: (0, 0)),
                  pl.BlockSpec((128, 128), lambda i: (0, 0)),
                  pl.BlockSpec((1, 128), lambda i: (0, 0)),
                  pl.BlockSpec((128, 128), lambda i: (0, 0)),
                  pl.BlockSpec((1, 128), lambda i: (0, 0)),
                  pl.BlockSpec((128, 4), lambda i: (0, 0)),
                  pl.BlockSpec((1, 4), lambda i: (0, 0))],
        out_specs=pl.BlockSpec((BM, 4), lambda i: (i, 0)),
        out_shape=jax.ShapeDtypeStruct((E4, 4), _f32),
    )(w4, t4, mom, fold, g.reshape(1, U), b.reshape(1, U),
      Wp0bd, bp0t, Wp1bd, bp1t, Wp2bd, bp2t)


def _edge_mm_body(w_ref, ew_ref, wbd_ref, b_ref, t_ref, mom_ref):
    i = pl.program_id(0)
    t = (jnp.dot(w_ref[...], wbd_ref[...], preferred_element_type=_f32)
         + b_ref[...] + ew_ref[...])
    t_ref[...] = t.astype(jnp.bfloat16)
    s1 = jnp.sum(t, axis=0, keepdims=True)
    s2 = jnp.sum(t * t, axis=0, keepdims=True)
    blk = jnp.concatenate([s1, s2], axis=0)

    @pl.when(i == 0)
    def _():
        mom_ref[...] = blk

    @pl.when(i != 0)
    def _():
        mom_ref[...] = mom_ref[...] + blk


def _tc_edge_mm(w4, ew4, Wbd, b128):
    return pl.pallas_call(
        _edge_mm_body,
        grid=(GE,),
        in_specs=[pl.BlockSpec((BM, 128), lambda i: (i, 0)),
                  pl.BlockSpec((BM, 128), lambda i: (i, 0)),
                  pl.BlockSpec((128, 128), lambda i: (0, 0)),
                  pl.BlockSpec((1, 128), lambda i: (0, 0))],
        out_specs=[pl.BlockSpec((BM, 128), lambda i: (i, 0)),
                   pl.BlockSpec((2, 128), lambda i: (0, 0))],
        out_shape=[jax.ShapeDtypeStruct((E4, 128), jnp.bfloat16),
                   jax.ShapeDtypeStruct((2, 128), _f32)],
    )(w4, ew4, Wbd, b128)


def _edge_upd_body(w_ref, t_ref, mom_ref, fold_ref, g_ref, b_ref, o_ref):
    s = jnp.dot(mom_ref[...], fold_ref[...], preferred_element_type=_f32)
    m = s[0:1, :] / E
    v = s[1:2, :] / E - m * m
    a = g_ref[...] / jnp.sqrt(v + EPS)
    cte = b_ref[...] - m * a
    ft = jnp.transpose(fold_ref[...], (1, 0))
    a128 = jnp.dot(a, ft, preferred_element_type=_f32)
    c128 = jnp.dot(cte, ft, preferred_element_type=_f32)
    tn = t_ref[...].astype(_f32) * a128 + c128
    o_ref[...] = w_ref[...] + _silu(tn)


def _tc_edge_update(w4, t4, mom, fold, g, b):
    return pl.pallas_call(
        _edge_upd_body,
        grid=(GE,),
        in_specs=[pl.BlockSpec((BM, 128), lambda i: (i, 0)),
                  pl.BlockSpec((BM, 128), lambda i: (i, 0)),
                  pl.BlockSpec((2, 128), lambda i: (0, 0)),
                  pl.BlockSpec((128, U), lambda i: (0, 0)),
                  pl.BlockSpec((1, U), lambda i: (0, 0)),
                  pl.BlockSpec((1, U), lambda i: (0, 0))],
        out_specs=pl.BlockSpec((BM, 128), lambda i: (i, 0)),
        out_shape=jax.ShapeDtypeStruct((E4, 128), _f32),
    )(w4, t4, mom, fold, g.reshape(1, U), b.reshape(1, U))


def _head_body(w_ref, w0_ref, b0_ref, w1_ref, b1_ref, w2_ref, b2_ref, o_ref):
    h = _silu(jnp.dot(w_ref[...], w0_ref[...], preferred_element_type=_f32)
              + b0_ref[...])
    h = _silu(jnp.dot(h, w1_ref[...], preferred_element_type=_f32)
              + b1_ref[...])
    t = jnp.dot(h, w2_ref[...], preferred_element_type=_f32) + b2_ref[...]
    o_ref[...] = 1.0 / (1.0 + jnp.exp(-t))


def _tc_head(w4, Wp0bd, bp0t, Wp1bd, bp1t, Wp2bd, bp2t):
    return pl.pallas_call(
        _head_body,
        grid=(GE,),
        in_specs=[pl.BlockSpec((BM, 128), lambda i: (i, 0)),
                  pl.BlockSpec((128, 128), lambda i: (0, 0)),
                  pl.BlockSpec((1, 128), lambda i: (0, 0)),
                  pl.BlockSpec((128, 128), lambda i: (0, 0)),
                  pl.BlockSpec((1, 128), lambda i: (0, 0)),
                  pl.BlockSpec((128, 4), lambda i: (0, 0)),
                  pl.BlockSpec((1, 4), lambda i: (0, 0))],
        out_specs=pl.BlockSpec((BM, 4), lambda i: (i, 0)),
        out_shape=jax.ShapeDtypeStruct((E4, 4), _f32),
    )(w4, Wp0bd, bp0t, Wp1bd, bp1t, Wp2bd, bp2t)


def _bd4(w):
    return jnp.kron(jnp.eye(4, dtype=_f32), w)


def kernel(x, edge_index, edge_attr, Wv0, bv0, We0, be0, Wv1, bv1, Wv2, bv2,
           Wv3, bv3, Wv4, bv4, We1, be1, vbn_g, vbn_b, ebn_g, ebn_b,
           Wp0, bp0, Wp1, bp1, Wp2, bp2):
    src = edge_index[0]
    dst = edge_index[1]
    zt = jnp.zeros((NP, U), dtype=_f32)
    on1 = jnp.ones((C, U), dtype=_f32)
    fold = jnp.tile(jnp.eye(U, dtype=_f32), (4, 1))

    x = _tc_init_x(x, Wv0, bv0)
    ea4 = edge_attr.reshape(E4, 12)
    w4 = _tc_init_w(ea4, _bd4(We0), jnp.tile(be0, 4).reshape(1, 128))
    (cntp,) = _sc_count(src, zt, on1)
    invc = _tc_invcnt(cntp)

    We1bd = jax.vmap(_bd4)(We1)
    be1t = jnp.tile(be1, (1, 4)).reshape(-1, 1, 128)

    D = Wv1.shape[0]
    x1, x2, x3, x4 = _tc_node_proj(x, Wv1[0], bv1[0], Wv2[0], bv2[0],
                                   Wv3[0], bv3[0], Wv4[0], bv4[0])
    for i in range(D):
        aggp, ewg = _sc_layer(w4.reshape(E, U), x2, x3, x4, src, dst, zt)
        if i + 1 < D:
            x, x1, x2, x3, x4 = _tc_node_step(
                x, x1, aggp, invc, vbn_g[i], vbn_b[i],
                Wv1[i + 1], bv1[i + 1], Wv2[i + 1], bv2[i + 1],
                Wv3[i + 1], bv3[i + 1], Wv4[i + 1], bv4[i + 1])
        t4, mom = _tc_edge_mm(w4, ewg.reshape(E4, 128), We1bd[i], be1t[i])
        if i + 1 < D:
            w4 = _tc_edge_update(w4, t4, mom, fold, ebn_g[i], ebn_b[i])
    heu4 = _tc_edge_tail(w4, t4, mom, fold, ebn_g[D - 1], ebn_b[D - 1],
                         _bd4(Wp0), jnp.tile(bp0, 4).reshape(1, 128),
                         _bd4(Wp1), jnp.tile(bp1, 4).reshape(1, 128),
                         _bd4(Wp2), jnp.tile(bp2, 4).reshape(1, 4))
    return heu4.reshape(E)


# speedup vs baseline: 1.0521x; 1.0002x over previous
import functools

import jax
import jax.numpy as jnp
from jax import lax
from jax.experimental import pallas as pl
from jax.experimental.pallas import tpu as pltpu
from jax.experimental.pallas import tpu_sc as plsc

N = 10000
E = 320000
U = 32
NC = 2
NS = 16
NW = NC * NS
EW = E // NW
C = 400
NCH = EW // C
NP = 10240
RPS = NP // NS
E4 = E // 4
BM = 8000
GE = E4 // BM
EPS = 1e-5

_f32 = jnp.float32


def _sc_layer_body(w0, x2t, x3t, x4t, src, dst, zt,
                   aggp, ew,
                   src_v0, src_v1, dst_v0, dst_v1,
                   w0_v0, w0_v1, x2r0, x2r1, x3r0, x3r1, x4r0, x4r1,
                   agg_s, sem_idx, semg0, semg1):
    src_v = (src_v0, src_v1)
    dst_v = (dst_v0, dst_v1)
    w0_v = (w0_v0, w0_v1)
    x2r = (x2r0, x2r1)
    x3r = (x3r0, x3r1)
    x4r = (x4r0, x4r1)
    semg = (semg0, semg1)

    c = lax.axis_index("c")
    s = lax.axis_index("s")
    wid = c * NS + s
    pltpu.sync_copy(zt.at[pl.ds(s * RPS, RPS)], agg_s.at[pl.ds(s * RPS, RPS)])
    plsc.subcore_barrier()

    def _issue(p, nb):
        base = pl.multiple_of(wid * EW + p * C, 8)
        ci1 = pltpu.async_copy(src.at[pl.ds(base, C)], src_v[nb], sem_idx)
        ci2 = pltpu.async_copy(dst.at[pl.ds(base, C)], dst_v[nb], sem_idx)
        ci1.wait()
        ci2.wait()
        pltpu.async_copy(w0.at[pl.ds(base, C)], w0_v[nb], semg[nb])
        pltpu.async_copy(x2t.at[dst_v[nb]], x2r[nb], semg[nb])
        pltpu.async_copy(x3t.at[src_v[nb]], x3r[nb], semg[nb])
        pltpu.async_copy(x4t.at[dst_v[nb]], x4r[nb], semg[nb])

    def _iter(p, b, issue_next):
        if issue_next:
            _issue(p + 1, 1 - b)
        pltpu.make_async_copy(w0.at[pl.ds(0, C)], w0_v[b], semg[b]).wait()
        pltpu.make_async_copy(w0.at[pl.ds(0, C)], x2r[b], semg[b]).wait()
        pltpu.make_async_copy(w0.at[pl.ds(0, C)], x3r[b], semg[b]).wait()
        pltpu.make_async_copy(w0.at[pl.ds(0, C)], x4r[b], semg[b]).wait()

        def row(r2, rc):
            for u in range(4):
                r = r2 * 4 + u
                for h in (0, 16):
                    wv = w0_v[b][r, pl.ds(h, 16)]
                    sig = 1.0 / (1.0 + jnp.exp(-wv))
                    x2r[b][r, pl.ds(h, 16)] = sig * x2r[b][r, pl.ds(h, 16)]
                    x3r[b][r, pl.ds(h, 16)] = (x3r[b][r, pl.ds(h, 16)]
                                               + x4r[b][r, pl.ds(h, 16)])
            return rc

        lax.fori_loop(0, C // 4, row, 0)
        base = pl.multiple_of(wid * EW + p * C, 8)
        pltpu.sync_copy(x2r[b], agg_s.at[src_v[b]], add=True)
        pltpu.sync_copy(x3r[b], ew.at[pl.ds(base, C)])

    _issue(0, 0)
    _iter(0, 0, True)

    def pair(q, carry):
        _iter(1 + 2 * q, 1, True)
        _iter(2 + 2 * q, 0, True)
        return carry

    lax.fori_loop(0, (NCH - 3) // 2, pair, 0)
    _iter(NCH - 2, 1, True)
    _iter(NCH - 1, 0, False)
    plsc.subcore_barrier()
    pltpu.sync_copy(agg_s.at[pl.ds(s * RPS, RPS)],
                    aggp.at[c, pl.ds(s * RPS, RPS)])


_sc_layer = functools.partial(
    pl.kernel,
    mesh=plsc.VectorSubcoreMesh(core_axis_name="c", subcore_axis_name="s"),
    compiler_params=pltpu.CompilerParams(use_tc_tiling_on_sc=False),
    out_type=[jax.ShapeDtypeStruct((NC, NP, U), _f32),
              jax.ShapeDtypeStruct((E, U), _f32)],
    scratch_types=(
        [pltpu.VMEM((C,), jnp.int32)] * 4
        + [pltpu.VMEM((C, U), _f32)] * 8
        + [pltpu.VMEM_SHARED((NP, U), _f32)]
        + [pltpu.SemaphoreType.DMA] * 3
    ),
)(_sc_layer_body)


def _sc_count_body(src, zt, on1, cntp, src_v, ones_v, cnt_s, sem):
    c = lax.axis_index("c")
    s = lax.axis_index("s")
    wid = c * NS + s
    pltpu.sync_copy(zt.at[pl.ds(s * RPS, RPS)], cnt_s.at[pl.ds(s * RPS, RPS)])
    pltpu.sync_copy(on1, ones_v)
    plsc.subcore_barrier()

    def chunk(k, carry):
        base = pl.multiple_of(wid * EW + k * C, 8)
        pltpu.sync_copy(src.at[pl.ds(base, C)], src_v)
        pltpu.sync_copy(ones_v, cnt_s.at[src_v], add=True)
        return carry

    lax.fori_loop(0, NCH, chunk, 0)
    plsc.subcore_barrier()
    pltpu.sync_copy(cnt_s.at[pl.ds(s * RPS, RPS)],
                    cntp.at[c, pl.ds(s * RPS, RPS)])


_sc_count = functools.partial(
    pl.kernel,
    mesh=plsc.VectorSubcoreMesh(core_axis_name="c", subcore_axis_name="s"),
    compiler_params=pltpu.CompilerParams(use_tc_tiling_on_sc=False),
    out_type=[jax.ShapeDtypeStruct((NC, NP, U), _f32)],
    scratch_types=[
        pltpu.VMEM((C,), jnp.int32),
        pltpu.VMEM((C, U), _f32),
        pltpu.VMEM_SHARED((NP, U), _f32),
        pltpu.SemaphoreType.DMA,
    ],
)(_sc_count_body)


def _silu(t):
    return t * (1.0 / (1.0 + jnp.exp(-t)))


def _init_x_body(x_ref, w_ref, b_ref, o_ref):
    t = x_ref[...] * w_ref[...] + b_ref[...]
    o_ref[...] = _silu(t)


def _tc_init_x(x, Wv0, bv0):
    return pl.pallas_call(
        _init_x_body,
        out_shape=jax.ShapeDtypeStruct((N, U), _f32),
    )(x, Wv0, bv0.reshape(1, U))


def _init_w_body(ea_ref, w_ref, b_ref, o_ref):
    t = jnp.dot(ea_ref[...], w_ref[...], preferred_element_type=_f32)
    o_ref[...] = _silu(t + b_ref[...])


def _tc_init_w(ea4, We0bd, be0t):
    return pl.pallas_call(
        _init_w_body,
        grid=(GE,),
        in_specs=[pl.BlockSpec((BM, 12), lambda i: (i, 0)),
                  pl.BlockSpec((12, 128), lambda i: (0, 0)),
                  pl.BlockSpec((1, 128), lambda i: (0, 0))],
        out_specs=pl.BlockSpec((BM, 128), lambda i: (i, 0)),
        out_shape=jax.ShapeDtypeStruct((E4, 128), _f32),
    )(ea4, We0bd, be0t)


def _node_proj_body(x_ref, w1, b1, w2, b2, w3, b3, w4, b4, o1, o2, o3, o4):
    xv = x_ref[...]
    o1[...] = jnp.dot(xv, w1[...], preferred_element_type=_f32) + b1[...]
    o2[...] = jnp.dot(xv, w2[...], preferred_element_type=_f32) + b2[...]
    o3[...] = jnp.dot(xv, w3[...], preferred_element_type=_f32) + b3[...]
    o4[...] = jnp.dot(xv, w4[...], preferred_element_type=_f32) + b4[...]


def _tc_node_proj(x, w1, b1, w2, b2, w3, b3, w4, b4):
    sh = jax.ShapeDtypeStruct((N, U), _f32)
    return pl.pallas_call(
        _node_proj_body,
        out_shape=[sh, sh, sh, sh],
    )(x, w1, b1.reshape(1, U), w2, b2.reshape(1, U),
      w3, b3.reshape(1, U), w4, b4.reshape(1, U))


def _invcnt_body(cntp_ref, o_ref):
    cnt = jnp.maximum(cntp_ref[0, :N, :] + cntp_ref[1, :N, :], 1.0)
    o_ref[...] = 1.0 / cnt


def _tc_invcnt(cntp):
    return pl.pallas_call(
        _invcnt_body,
        out_shape=jax.ShapeDtypeStruct((N, U), _f32),
    )(cntp)


def _node_step_body(x_ref, x1_ref, aggp_ref, invc_ref, g_ref, b_ref,
                    w1, b1, w2, b2, w3, b3, w4, b4,
                    ox, o1, o2, o3, o4):
    agg = (aggp_ref[0, :N, :] + aggp_ref[1, :N, :]) * invc_ref[...]
    t = x1_ref[...] + agg
    m = jnp.mean(t, axis=0, keepdims=True)
    v = jnp.mean((t - m) * (t - m), axis=0, keepdims=True)
    tn = (t - m) / jnp.sqrt(v + EPS) * g_ref[...] + b_ref[...]
    xn = x_ref[...] + _silu(tn)
    ox[...] = xn
    o1[...] = jnp.dot(xn, w1[...], preferred_element_type=_f32) + b1[...]
    o2[...] = jnp.dot(xn, w2[...], preferred_element_type=_f32) + b2[...]
    o3[...] = jnp.dot(xn, w3[...], preferred_element_type=_f32) + b3[...]
    o4[...] = jnp.dot(xn, w4[...], preferred_element_type=_f32) + b4[...]


def _tc_node_step(x, x1, aggp, invc, g, b, w1, b1, w2, b2, w3, b3, w4, b4):
    sh = jax.ShapeDtypeStruct((N, U), _f32)
    return pl.pallas_call(
        _node_step_body,
        out_shape=[sh, sh, sh, sh, sh],
    )(x, x1, aggp, invc, g.reshape(1, U), b.reshape(1, U),
      w1, b1.reshape(1, U), w2, b2.reshape(1, U),
      w3, b3.reshape(1, U), w4, b4.reshape(1, U))


def _edge_tail_body(w_ref, t_ref, mom_ref, fold_ref, g_ref, b_ref,
                    wp0, bp0r, wp1, bp1r, wp2, bp2r, o_ref):
    s = jnp.dot(mom_ref[...], fold_ref[...], preferred_element_type=_f32)
    m = s[0:1, :] / E
    v = s[1:2, :] / E - m * m
    a = g_ref[...] / jnp.sqrt(v + EPS)
    cte = b_ref[...] - m * a
    ft = jnp.transpose(fold_ref[...], (1, 0))
    a128 = jnp.dot(a, ft, preferred_element_type=_f32)
    c128 = jnp.dot(cte, ft, preferred_element_type=_f32)
    wn = w_ref[...] + _silu(t_ref[...].astype(_f32) * a128 + c128)
    h = _silu(jnp.dot(wn, wp0[...], preferred_element_type=_f32) + bp0r[...])
    h = _silu(jnp.dot(h, wp1[...], preferred_element_type=_f32) + bp1r[...])
    t = jnp.dot(h, wp2[...], preferred_element_type=_f32) + bp2r[...]
    o_ref[...] = 1.0 / (1.0 + jnp.exp(-t))


def _tc_edge_tail(w4, t4, mom, fold, g, b, Wp0bd, bp0t, Wp1bd, bp1t, Wp2bd, bp2t):
    return pl.pallas_call(
        _edge_tail_body,
        grid=(GE,),
        in_specs=[pl.BlockSpec((BM, 128), lambda i: (i, 0)),
                  pl.BlockSpec((BM, 128), lambda i: (i, 0)),
                  pl.BlockSpec((2, 128), lambda i: (0, 0)),
                  pl.BlockSpec((128, U), lambda i: (0, 0)),
                  pl.BlockSpec((1, U), lambda i: (0, 0)),
                  pl.BlockSpec((1, U), lambda i: (0, 0)),
                  pl.BlockSpec((128, 128), lambda i: (0, 0)),
                  pl.BlockSpec((1, 128), lambda i: (0, 0)),
                  pl.BlockSpec((128, 128), lambda i: (0, 0)),
                  pl.BlockSpec((1, 128), lambda i: (0, 0)),
                  pl.BlockSpec((128, 4), lambda i: (0, 0)),
                  pl.BlockSpec((1, 4), lambda i: (0, 0))],
        out_specs=pl.BlockSpec((BM, 4), lambda i: (i, 0)),
        out_shape=jax.ShapeDtypeStruct((E4, 4), _f32),
    )(w4, t4, mom, fold, g.reshape(1, U), b.reshape(1, U),
      Wp0bd, bp0t, Wp1bd, bp1t, Wp2bd, bp2t)


def _edge_mm_body(w_ref, ew_ref, wbd_ref, b_ref, t_ref, mom_ref):
    i = pl.program_id(0)
    t = (jnp.dot(w_ref[...], wbd_ref[...], preferred_element_type=_f32)
         + b_ref[...] + ew_ref[...])
    t_ref[...] = t.astype(jnp.bfloat16)
    s1 = jnp.sum(t, axis=0, keepdims=True)
    s2 = jnp.sum(t * t, axis=0, keepdims=True)
    blk = jnp.concatenate([s1, s2], axis=0)

    @pl.when(i == 0)
    def _():
        mom_ref[...] = blk

    @pl.when(i != 0)
    def _():
        mom_ref[...] = mom_ref[...] + blk


def _tc_edge_mm(w4, ew4, Wbd, b128):
    return pl.pallas_call(
        _edge_mm_body,
        grid=(GE,),
        in_specs=[pl.BlockSpec((BM, 128), lambda i: (i, 0)),
                  pl.BlockSpec((BM, 128), lambda i: (i, 0)),
                  pl.BlockSpec((128, 128), lambda i: (0, 0)),
                  pl.BlockSpec((1, 128), lambda i: (0, 0))],
        out_specs=[pl.BlockSpec((BM, 128), lambda i: (i, 0)),
                   pl.BlockSpec((2, 128), lambda i: (0, 0))],
        out_shape=[jax.ShapeDtypeStruct((E4, 128), jnp.bfloat16),
                   jax.ShapeDtypeStruct((2, 128), _f32)],
    )(w4, ew4, Wbd, b128)


def _edge_upd_body(w_ref, t_ref, mom_ref, fold_ref, g_ref, b_ref, o_ref):
    s = jnp.dot(mom_ref[...], fold_ref[...], preferred_element_type=_f32)
    m = s[0:1, :] / E
    v = s[1:2, :] / E - m * m
    a = g_ref[...] / jnp.sqrt(v + EPS)
    cte = b_ref[...] - m * a
    ft = jnp.transpose(fold_ref[...], (1, 0))
    a128 = jnp.dot(a, ft, preferred_element_type=_f32)
    c128 = jnp.dot(cte, ft, preferred_element_type=_f32)
    tn = t_ref[...].astype(_f32) * a128 + c128
    o_ref[...] = w_ref[...] + _silu(tn)


def _tc_edge_update(w4, t4, mom, fold, g, b):
    return pl.pallas_call(
        _edge_upd_body,
        grid=(GE,),
        in_specs=[pl.BlockSpec((BM, 128), lambda i: (i, 0)),
                  pl.BlockSpec((BM, 128), lambda i: (i, 0)),
                  pl.BlockSpec((2, 128), lambda i: (0, 0)),
                  pl.BlockSpec((128, U), lambda i: (0, 0)),
                  pl.BlockSpec((1, U), lambda i: (0, 0)),
                  pl.BlockSpec((1, U), lambda i: (0, 0))],
        out_specs=pl.BlockSpec((BM, 128), lambda i: (i, 0)),
        out_shape=jax.ShapeDtypeStruct((E4, 128), _f32),
    )(w4, t4, mom, fold, g.reshape(1, U), b.reshape(1, U))


def _bd4(w):
    return jnp.kron(jnp.eye(4, dtype=_f32), w)


def kernel(x, edge_index, edge_attr, Wv0, bv0, We0, be0, Wv1, bv1, Wv2, bv2,
           Wv3, bv3, Wv4, bv4, We1, be1, vbn_g, vbn_b, ebn_g, ebn_b,
           Wp0, bp0, Wp1, bp1, Wp2, bp2):
    src = edge_index[0]
    dst = edge_index[1]
    zt = jnp.zeros((NP, U), dtype=_f32)
    on1 = jnp.ones((C, U), dtype=_f32)
    fold = jnp.tile(jnp.eye(U, dtype=_f32), (4, 1))

    x = _tc_init_x(x, Wv0, bv0)
    ea4 = edge_attr.reshape(E4, 12)
    w4 = _tc_init_w(ea4, _bd4(We0), jnp.tile(be0, 4).reshape(1, 128))
    (cntp,) = _sc_count(src, zt, on1)
    invc = _tc_invcnt(cntp)

    We1bd = jax.vmap(_bd4)(We1)
    be1t = jnp.tile(be1, (1, 4)).reshape(-1, 1, 128)

    D = Wv1.shape[0]
    x1, x2, x3, x4 = _tc_node_proj(x, Wv1[0], bv1[0], Wv2[0], bv2[0],
                                   Wv3[0], bv3[0], Wv4[0], bv4[0])
    for i in range(D):
        aggp, ewg = _sc_layer(w4.reshape(E, U), x2, x3, x4, src, dst, zt)
        if i + 1 < D:
            x, x1, x2, x3, x4 = _tc_node_step(
                x, x1, aggp, invc, vbn_g[i], vbn_b[i],
                Wv1[i + 1], bv1[i + 1], Wv2[i + 1], bv2[i + 1],
                Wv3[i + 1], bv3[i + 1], Wv4[i + 1], bv4[i + 1])
        t4, mom = _tc_edge_mm(w4, ewg.reshape(E4, 128), We1bd[i], be1t[i])
        if i + 1 < D:
            w4 = _tc_edge_update(w4, t4, mom, fold, ebn_g[i], ebn_b[i])
    heu4 = _tc_edge_tail(w4, t4, mom, fold, ebn_g[D - 1], ebn_b[D - 1],
                         _bd4(Wp0), jnp.tile(bp0, 4).reshape(1, 128),
                         _bd4(Wp1), jnp.tile(bp1, 4).reshape(1, 128),
                         _bd4(Wp2), jnp.tile(bp2, 4).reshape(1, 4))
    return heu4.reshape(E)
